# baseline (speedup 1.0000x reference)
import dataclasses
import functools

import jax
import jax.numpy as jnp
import numpy as np
from jax import lax
from jax.experimental import pallas as pl
from jax.experimental.pallas import tpu as pltpu
from jax.experimental.pallas import tpu_sc as plsc

_B, _S, _D, _OUT, _E, _K = 1, 2048, 768, 768, 8, 2
_T = _B * _S
_NP = _T * _K
_TM = 256
_NT = _T // _TM
_TS = 128
_P = _NP + _E * _TS
_NTS = _P // _TS
_NC, _NS = 2, 16
_NW = _NC * _NS


def _sc_compiler_params():
    cp = pltpu.CompilerParams()
    if "needs_layout_passes" in pltpu.CompilerParams.__dataclass_fields__:
        cp = dataclasses.replace(cp, needs_layout_passes=False)
    return cp


def _gate_body(x_ref, wg1_ref, wg2_ref, pos_ref, s1_ref, s2_ref, te_ref,
               logit_scr, carry_scr, astart_scr, lt_scr):
    p = pl.program_id(0)
    t = pl.program_id(1)

    @pl.when((p == 0) & (t == 0))
    def _init():
        carry_scr[...] = jnp.zeros((1, _E), jnp.float32)
        npt = 2 * _TM
        r_row = lax.broadcasted_iota(jnp.int32, (npt, npt), 0)
        r_col = lax.broadcasted_iota(jnp.int32, (npt, npt), 1)
        lt_scr[...] = (r_row > r_col).astype(jnp.bfloat16)

    @pl.when(p == 0)
    def _pass0():
        x = x_ref[...]
        h = jnp.tanh(
            lax.dot_general(x, wg1_ref[...], (((1,), (1,)), ((), ())),
                            preferred_element_type=jnp.float32))
        logits = lax.dot_general(h, wg2_ref[...], (((1,), (1,)), ((), ())),
                                 preferred_element_type=jnp.float32)
        logit_scr[pl.ds(t * _TM, _TM), 0:_E] = logits
        col = lax.broadcasted_iota(jnp.int32, (_TM, _E), 1)
        i1 = jnp.argmax(logits, axis=1)[:, None]
        masked = jnp.where(col == i1, -jnp.inf, logits)
        i2 = jnp.argmax(masked, axis=1)[:, None]
        onehot = jnp.concatenate(
            [(col == i1).astype(jnp.float32), (col == i2).astype(jnp.float32)],
            axis=0)
        carry_scr[...] += jnp.sum(onehot, axis=0, keepdims=True)

    @pl.when((p == 1) & (t == 0))
    def _starts():
        counts = carry_scr[...]
        padded = jnp.floor((counts + (_TS - 1)) / _TS) * _TS
        e_row = lax.broadcasted_iota(jnp.int32, (_E, _E), 0)
        e_col = lax.broadcasted_iota(jnp.int32, (_E, _E), 1)
        ltri = (e_row < e_col).astype(jnp.float32)
        astart = lax.dot_general(padded, ltri, (((1,), (0,)), ((), ())),
                                 preferred_element_type=jnp.float32)
        astart_scr[...] = astart
        carry_scr[...] = jnp.zeros((1, _E), jnp.float32)
        aend = astart + padded
        m_iota = (lax.broadcasted_iota(jnp.int32, (64, _E), 0)
                  .astype(jnp.float32) * _TS)
        te = jnp.sum((m_iota >= jnp.broadcast_to(aend, (64, _E)))
                     .astype(jnp.int32), axis=1)
        te_ref[0, 0, :] = jnp.minimum(te, _E - 1)

    @pl.when(p == 1)
    def _pass1():
        logits = logit_scr[pl.ds(t * _TM, _TM), 0:_E]
        col = lax.broadcasted_iota(jnp.int32, (_TM, _E), 1)
        m1 = jnp.max(logits, axis=1, keepdims=True)
        i1 = jnp.argmax(logits, axis=1)[:, None]
        masked = jnp.where(col == i1, -jnp.inf, logits)
        m2 = jnp.max(masked, axis=1, keepdims=True)
        i2 = jnp.argmax(masked, axis=1)[:, None]
        s1 = 1.0 / (1.0 + jnp.exp(m2 - m1))
        s1_ref[0, 0, :] = s1[:, 0]
        s2_ref[0, 0, :] = 1.0 - s1[:, 0]
        onehot = jnp.concatenate(
            [(col == i1).astype(jnp.float32), (col == i2).astype(jnp.float32)],
            axis=0)
        rank_local = lax.dot_general(
            lt_scr[...], onehot.astype(jnp.bfloat16),
            (((1,), (0,)), ((), ())), preferred_element_type=jnp.float32)
        base = carry_scr[...] + astart_scr[...]
        pos = jnp.sum(onehot * (rank_local + base), axis=1)
        posi = pos.astype(jnp.int32)
        pos_ref[0, 0, :] = posi[:_TM]
        pos_ref[0, 1, :] = posi[_TM:]
        carry_scr[...] += jnp.sum(onehot, axis=0, keepdims=True)


def _run_gate(xf, Wg1, Wg2):
    return pl.pallas_call(
        _gate_body,
        grid=(2, _NT),
        in_specs=[
            pl.BlockSpec((_TM, _D), lambda p, t: (t * (1 - p), 0)),
            pl.BlockSpec((_E, _D), lambda p, t: (0, 0)),
            pl.BlockSpec((_E, _E), lambda p, t: (0, 0)),
        ],
        out_specs=[
            pl.BlockSpec((1, _K, _TM), lambda p, t: (t, 0, 0)),
            pl.BlockSpec((1, 1, _TM), lambda p, t: (t, 0, 0)),
            pl.BlockSpec((1, 1, _TM), lambda p, t: (t, 0, 0)),
            pl.BlockSpec((1, 1, 64), lambda p, t: (0, 0, 0)),
        ],
        out_shape=[
            jax.ShapeDtypeStruct((_NT, _K, _TM), jnp.int32),
            jax.ShapeDtypeStruct((_NT, 1, _TM), jnp.float32),
            jax.ShapeDtypeStruct((_NT, 1, _TM), jnp.float32),
            jax.ShapeDtypeStruct((1, 1, 64), jnp.int32),
        ],
        scratch_shapes=[
            pltpu.VMEM((_T, 128), jnp.float32),
            pltpu.VMEM((1, _E), jnp.float32),
            pltpu.VMEM((1, _E), jnp.float32),
            pltpu.VMEM((2 * _TM, 2 * _TM), jnp.bfloat16),
        ],
    )(xf, Wg1, Wg2)


def _sc_dispatch(xf, pos, tok):
    pairs_per_w = _NP // _NW
    mesh = plsc.VectorSubcoreMesh(core_axis_name="c", subcore_axis_name="s")

    @functools.partial(
        pl.kernel, mesh=mesh,
        out_type=jax.ShapeDtypeStruct((_P, _D), jnp.float32),
        scratch_types=[
            pltpu.VMEM((pairs_per_w,), jnp.int32),
            pltpu.VMEM((pairs_per_w,), jnp.int32),
            pltpu.VMEM((pairs_per_w, _D), jnp.float32),
            pltpu.SemaphoreType.DMA,
        ],
        compiler_params=_sc_compiler_params(),
    )
    def k(x_hbm, pos_hbm, tok_hbm, xs_hbm, pos_v, tok_v, rows_v, sem):
        wid = lax.axis_index("s") * _NC + lax.axis_index("c")
        base = wid * pairs_per_w
        pltpu.sync_copy(pos_hbm.at[pl.ds(base, pairs_per_w)], pos_v)
        pltpu.sync_copy(tok_hbm.at[pl.ds(base, pairs_per_w)], tok_v)
        pltpu.async_copy(x_hbm.at[tok_v], rows_v, sem).wait()
        pltpu.async_copy(rows_v, xs_hbm.at[pos_v], sem).wait()

    return k(xf, pos, tok)


def _mm_body(te_ref, xs_ref, w_ref, b_ref, ys_ref):
    xb = xs_ref[...].astype(jnp.bfloat16)
    ys_ref[...] = lax.dot_general(
        xb, w_ref[0], (((1,), (1,)), ((), ())),
        preferred_element_type=jnp.float32) + b_ref[0]


def _run_grouped_mm(te, Xs, W, b):
    grid_spec = pltpu.PrefetchScalarGridSpec(
        num_scalar_prefetch=1,
        grid=(_NTS,),
        in_specs=[
            pl.BlockSpec((_TS, _D), lambda i, s: (i, 0)),
            pl.BlockSpec((1, _OUT, _D), lambda i, s: (s[i], 0, 0)),
            pl.BlockSpec((1, 1, _OUT), lambda i, s: (s[i], 0, 0)),
        ],
        out_specs=pl.BlockSpec((_TS, _OUT), lambda i, s: (i, 0)),
    )
    return pl.pallas_call(
        _mm_body,
        grid_spec=grid_spec,
        out_shape=jax.ShapeDtypeStruct((_P, _OUT), jnp.float32),
    )(te, Xs, W.astype(jnp.bfloat16), b.reshape(_E, 1, _OUT))


def _sc_combine_gather(Ys, posA, posB):
    toks_per_w = _T // _NW
    mesh = plsc.VectorSubcoreMesh(core_axis_name="c", subcore_axis_name="s")

    @functools.partial(
        pl.kernel, mesh=mesh,
        out_type=[jax.ShapeDtypeStruct((_T, _OUT), jnp.float32),
                  jax.ShapeDtypeStruct((_T, _OUT), jnp.float32)],
        scratch_types=[
            pltpu.VMEM((toks_per_w,), jnp.int32),
            pltpu.VMEM((toks_per_w, _OUT), jnp.float32),
            pltpu.SemaphoreType.DMA,
        ],
        compiler_params=_sc_compiler_params(),
    )
    def k(ys_hbm, pa_hbm, pb_hbm, oa_hbm, ob_hbm, idx_v, rows_v, sem):
        wid = lax.axis_index("s") * _NC + lax.axis_index("c")
        base = wid * toks_per_w
        pltpu.sync_copy(pa_hbm.at[pl.ds(base, toks_per_w)], idx_v)
        pltpu.async_copy(ys_hbm.at[idx_v], rows_v, sem).wait()
        pltpu.sync_copy(rows_v, oa_hbm.at[pl.ds(base, toks_per_w)])
        pltpu.sync_copy(pb_hbm.at[pl.ds(base, toks_per_w)], idx_v)
        pltpu.async_copy(ys_hbm.at[idx_v], rows_v, sem).wait()
        pltpu.sync_copy(rows_v, ob_hbm.at[pl.ds(base, toks_per_w)])

    return k(Ys, posA, posB)


def _comb_body(a_ref, b_ref, s1_ref, s2_ref, y_ref):
    y_ref[...] = s1_ref[...] * a_ref[...] + s2_ref[...] * b_ref[...]


def _run_combine(YsA, YsB, s1c, s2c):
    return pl.pallas_call(
        _comb_body,
        grid=(_NT,),
        in_specs=[
            pl.BlockSpec((_TM, _OUT), lambda i: (i, 0)),
            pl.BlockSpec((_TM, _OUT), lambda i: (i, 0)),
            pl.BlockSpec((_TM, 1), lambda i: (i, 0)),
            pl.BlockSpec((_TM, 1), lambda i: (i, 0)),
        ],
        out_specs=pl.BlockSpec((_TM, _OUT), lambda i: (i, 0)),
        out_shape=jax.ShapeDtypeStruct((_T, _OUT), jnp.float32),
    )(YsA, YsB, s1c, s2c)


def kernel(x, Wg1, Wg2, W, b):
    bs, sl, d = x.shape
    xf = x.reshape(-1, d)
    pos, s1o, s2o, te = _run_gate(xf, Wg1, Wg2)
    pair = np.arange(_NP)
    tok_const = jnp.asarray((pair // (_K * _TM)) * _TM + pair % _TM,
                            dtype=jnp.int32)
    Xs = _sc_dispatch(xf, pos.reshape(_NP), tok_const)
    Ys = _run_grouped_mm(te.reshape(64)[:_NTS], Xs, W, b)
    posA = pos[:, 0, :].reshape(_T)
    posB = pos[:, 1, :].reshape(_T)
    YsA, YsB = _sc_combine_gather(Ys, posA, posB)
    y = _run_combine(YsA, YsB, s1o.reshape(_T, 1), s2o.reshape(_T, 1))
    return y.reshape(bs, sl, _OUT), jnp.float32(-100.0)


# speedup vs baseline: 3.3598x; 3.3598x over previous
import jax
import jax.numpy as jnp
from jax import lax
from jax.experimental import pallas as pl
from jax.experimental.pallas import tpu as pltpu

_B, _S, _D, _OUT, _E, _K = 1, 2048, 768, 768, 8, 2
_TM = 256


def _moe_body(x_ref, wg1_ref, wg2_ref, w_ref, b_ref, y_ref):
    x = x_ref[...]
    h = jnp.tanh(
        lax.dot_general(x, wg1_ref[...], (((1,), (1,)), ((), ())),
                        preferred_element_type=jnp.float32))
    logits = lax.dot_general(h, wg2_ref[...], (((1,), (1,)), ((), ())),
                             preferred_element_type=jnp.float32)
    m1 = jnp.max(logits, axis=1, keepdims=True)
    col = lax.broadcasted_iota(jnp.int32, (_TM, _E), 1)
    i1 = jnp.argmax(logits, axis=1)[:, None]
    masked = jnp.where(col == i1, -jnp.inf, logits)
    m2 = jnp.max(masked, axis=1, keepdims=True)
    i2 = jnp.argmax(masked, axis=1)[:, None]
    s1 = 1.0 / (1.0 + jnp.exp(m2 - m1))
    s2 = 1.0 - s1
    combine = (jnp.where(col == i1, s1, 0.0)
               + jnp.where(col == i2, s2, 0.0))
    acc = lax.dot_general(combine, b_ref[...], (((1,), (0,)), ((), ())),
                          preferred_element_type=jnp.float32)
    for e in range(_E):
        ye = lax.dot_general(x, w_ref[e], (((1,), (1,)), ((), ())),
                             preferred_element_type=jnp.float32)
        acc = acc + combine[:, e:e + 1] * ye
    y_ref[...] = acc


def kernel(x, Wg1, Wg2, W, b):
    bs, sl, d = x.shape
    xf = x.reshape(-1, d)
    T = xf.shape[0]
    y = pl.pallas_call(
        _moe_body,
        grid=(T // _TM,),
        in_specs=[
            pl.BlockSpec((_TM, _D), lambda i: (i, 0)),
            pl.BlockSpec((_E, _D), lambda i: (0, 0)),
            pl.BlockSpec((_E, _E), lambda i: (0, 0)),
            pl.BlockSpec((_E, _OUT, _D), lambda i: (0, 0, 0)),
            pl.BlockSpec((_E, _OUT), lambda i: (0, 0)),
        ],
        out_specs=pl.BlockSpec((_TM, _OUT), lambda i: (i, 0)),
        out_shape=jax.ShapeDtypeStruct((T, _OUT), jnp.float32),
    )(xf, Wg1, Wg2, W, b)
    return y.reshape(bs, sl, _OUT), jnp.float32(-100.0)


# speedup vs baseline: 3.3999x; 1.0119x over previous
import jax
import jax.numpy as jnp
from jax import lax
from jax.experimental import pallas as pl
from jax.experimental.pallas import tpu as pltpu

_B, _S, _D, _OUT, _E, _K = 1, 2048, 768, 768, 8, 2
_TM = 512


def _moe_body(x_ref, wg1_ref, wg2_ref, w_ref, b_ref, y_ref):
    x = x_ref[...]
    h = jnp.tanh(
        lax.dot_general(x, wg1_ref[...], (((1,), (1,)), ((), ())),
                        preferred_element_type=jnp.float32))
    logits = lax.dot_general(h, wg2_ref[...], (((1,), (1,)), ((), ())),
                             preferred_element_type=jnp.float32)
    m1 = jnp.max(logits, axis=1, keepdims=True)
    col = lax.broadcasted_iota(jnp.int32, (_TM, _E), 1)
    i1 = jnp.argmax(logits, axis=1)[:, None]
    masked = jnp.where(col == i1, -jnp.inf, logits)
    m2 = jnp.max(masked, axis=1, keepdims=True)
    i2 = jnp.argmax(masked, axis=1)[:, None]
    s1 = 1.0 / (1.0 + jnp.exp(m2 - m1))
    s2 = 1.0 - s1
    combine = (jnp.where(col == i1, s1, 0.0)
               + jnp.where(col == i2, s2, 0.0))
    acc = lax.dot_general(combine, b_ref[...], (((1,), (0,)), ((), ())),
                          preferred_element_type=jnp.float32)
    for e in range(_E):
        ye = lax.dot_general(x, w_ref[e], (((1,), (1,)), ((), ())),
                             preferred_element_type=jnp.float32)
        acc = acc + combine[:, e:e + 1] * ye
    y_ref[...] = acc


def kernel(x, Wg1, Wg2, W, b):
    bs, sl, d = x.shape
    xf = x.reshape(-1, d)
    T = xf.shape[0]
    y = pl.pallas_call(
        _moe_body,
        grid=(T // _TM,),
        in_specs=[
            pl.BlockSpec((_TM, _D), lambda i: (i, 0)),
            pl.BlockSpec((_E, _D), lambda i: (0, 0)),
            pl.BlockSpec((_E, _E), lambda i: (0, 0)),
            pl.BlockSpec((_E, _OUT, _D), lambda i: (0, 0, 0)),
            pl.BlockSpec((_E, _OUT), lambda i: (0, 0)),
        ],
        out_specs=pl.BlockSpec((_TM, _OUT), lambda i: (i, 0)),
        out_shape=jax.ShapeDtypeStruct((T, _OUT), jnp.float32),
    )(xf, Wg1, Wg2, W, b)
    return y.reshape(bs, sl, _OUT), jnp.float32(-100.0)
